# initial kernel scaffold (unmeasured)
import jax
import jax.numpy as jnp
from jax import lax
from jax.experimental import pallas as pl
from jax.experimental.pallas import tpu as pltpu

N_DEV = 8
NBITS = 10


def kernel(x, dest):
    t, d_model = x.shape

    order = jnp.argsort(dest, stable=True)
    x_sorted = jnp.take(x, order, axis=0)
    counts = jnp.bincount(dest, length=N_DEV).astype(jnp.int32).reshape(1, N_DEV)

    def body(
        x_ref,
        counts_ref,
        out_ref,
        cnt_vmem,
        cnt_smem,
        cnt_send_sems,
        cnt_recv_sems,
        d_send_sems,
        d_recv_sems,
        copy_sem,
    ):
        me = lax.axis_index("i")

        barrier = pltpu.get_barrier_semaphore()
        for o in range(1, N_DEV):
            pl.semaphore_signal(
                barrier,
                inc=1,
                device_id=((me + o) % N_DEV,),
                device_id_type=pl.DeviceIdType.MESH,
            )
        pl.semaphore_wait(barrier, N_DEV - 1)

        own = pltpu.make_async_copy(
            counts_ref, cnt_vmem.at[pl.ds(me, 1)], copy_sem
        )
        own.start()
        own.wait()
        for o in range(1, N_DEV):
            rdma = pltpu.make_async_remote_copy(
                src_ref=counts_ref,
                dst_ref=cnt_vmem.at[pl.ds(me, 1)],
                send_sem=cnt_send_sems.at[o],
                recv_sem=cnt_recv_sems.at[o],
                device_id=((me + o) % N_DEV,),
                device_id_type=pl.DeviceIdType.MESH,
            )
            rdma.start()
        for o in range(1, N_DEV):
            pltpu.make_async_remote_copy(
                src_ref=counts_ref,
                dst_ref=cnt_vmem.at[pl.ds(0, 1)],
                send_sem=cnt_send_sems.at[o],
                recv_sem=cnt_recv_sems.at[o],
                device_id=(0,),
                device_id_type=pl.DeviceIdType.MESH,
            ).wait_recv()
        for o in range(1, N_DEV):
            pltpu.make_async_remote_copy(
                src_ref=counts_ref,
                dst_ref=cnt_vmem.at[pl.ds(0, 1)],
                send_sem=cnt_send_sems.at[o],
                recv_sem=cnt_recv_sems.at[o],
                device_id=(0,),
                device_id_type=pl.DeviceIdType.MESH,
            ).wait_send()

        cp = pltpu.make_async_copy(cnt_vmem, cnt_smem, copy_sem)
        cp.start()
        cp.wait()

        def cnt(s, d):
            return cnt_smem[s, d]

        def col_off_before(src, dst):
            acc = jnp.int32(0)
            for s in range(N_DEV):
                acc = acc + jnp.where(s < src, cnt(s, dst), 0)
            return acc

        def row_off_before(src, dst):
            acc = jnp.int32(0)
            for d in range(N_DEV):
                acc = acc + jnp.where(d < dst, cnt(src, d), 0)
            return acc

        for o in range(1, N_DEV):
            dst_rank = (me + o) % N_DEV
            c = cnt(me, dst_rank)
            src0 = row_off_before(me, dst_rank)
            dst0 = col_off_before(me, dst_rank)
            done = jnp.int32(0)
            for k in range(NBITS - 1, -1, -1):
                size = 1 << k
                bit = (c // size) % 2

                @pl.when(bit == 1)
                def _(o=o, k=k, size=size, src0=src0, dst0=dst0, done=done,
                      dst_rank=dst_rank):
                    pltpu.make_async_remote_copy(
                        src_ref=x_ref.at[pl.ds(src0 + done, size)],
                        dst_ref=out_ref.at[pl.ds(dst0 + done, size)],
                        send_sem=d_send_sems.at[o, k],
                        recv_sem=d_recv_sems.at[o, k],
                        device_id=(dst_rank,),
                        device_id_type=pl.DeviceIdType.MESH,
                    ).start()

                done = done + bit * size

        c = cnt(me, me)
        src0 = row_off_before(me, me)
        dst0 = col_off_before(me, me)
        done = jnp.int32(0)
        for k in range(NBITS - 1, -1, -1):
            size = 1 << k
            bit = (c // size) % 2

            @pl.when(bit == 1)
            def _(size=size, src0=src0, dst0=dst0, done=done):
                local = pltpu.make_async_copy(
                    x_ref.at[pl.ds(src0 + done, size)],
                    out_ref.at[pl.ds(dst0 + done, size)],
                    copy_sem,
                )
                local.start()
                local.wait()

            done = done + bit * size

        for o in range(1, N_DEV):
            src_rank = (me - o) % N_DEV
            c_in = cnt(src_rank, me)
            for k in range(NBITS - 1, -1, -1):
                size = 1 << k
                bit = (c_in // size) % 2

                @pl.when(bit == 1)
                def _(o=o, k=k, size=size):
                    pltpu.make_async_remote_copy(
                        src_ref=x_ref.at[pl.ds(0, size)],
                        dst_ref=out_ref.at[pl.ds(0, size)],
                        send_sem=d_send_sems.at[o, k],
                        recv_sem=d_recv_sems.at[o, k],
                        device_id=(0,),
                        device_id_type=pl.DeviceIdType.MESH,
                    ).wait_recv()

        for o in range(1, N_DEV):
            dst_rank = (me + o) % N_DEV
            c_out = cnt(me, dst_rank)
            for k in range(NBITS - 1, -1, -1):
                size = 1 << k
                bit = (c_out // size) % 2

                @pl.when(bit == 1)
                def _(o=o, k=k, size=size):
                    pltpu.make_async_remote_copy(
                        src_ref=x_ref.at[pl.ds(0, size)],
                        dst_ref=out_ref.at[pl.ds(0, size)],
                        send_sem=d_send_sems.at[o, k],
                        recv_sem=d_recv_sems.at[o, k],
                        device_id=(0,),
                        device_id_type=pl.DeviceIdType.MESH,
                    ).wait_send()

    return pl.pallas_call(
        body,
        out_shape=jax.ShapeDtypeStruct((t, d_model), jnp.float32),
        in_specs=[
            pl.BlockSpec(memory_space=pltpu.VMEM),
            pl.BlockSpec(memory_space=pltpu.VMEM),
        ],
        out_specs=pl.BlockSpec(memory_space=pltpu.VMEM),
        scratch_shapes=[
            pltpu.VMEM((N_DEV, N_DEV), jnp.int32),
            pltpu.SMEM((N_DEV, N_DEV), jnp.int32),
            pltpu.SemaphoreType.DMA((N_DEV,)),
            pltpu.SemaphoreType.DMA((N_DEV,)),
            pltpu.SemaphoreType.DMA((N_DEV, NBITS)),
            pltpu.SemaphoreType.DMA((N_DEV, NBITS)),
            pltpu.SemaphoreType.DMA,
        ],
        compiler_params=pltpu.CompilerParams(collective_id=0),
    )(x_sorted, counts)


# baseline (device time: 41585 ns/iter reference)
import jax
import jax.numpy as jnp
from jax import lax
from jax.experimental import pallas as pl
from jax.experimental.pallas import tpu as pltpu

N_DEV = 8
NBITS = 10


def kernel(x, dest):
    t, d_model = x.shape

    order = jnp.argsort(dest, stable=True)
    x_sorted = jnp.take(x, order, axis=0).reshape(-1)
    counts = jnp.bincount(dest, length=N_DEV).astype(jnp.int32).reshape(1, N_DEV)

    def rows(ref, row0, nrows):
        if isinstance(row0, int):
            return ref.at[pl.ds(row0 * d_model, nrows * d_model)]
        return ref.at[pl.ds(pl.multiple_of(row0 * d_model, d_model),
                            nrows * d_model)]

    def body(
        x_ref,
        counts_ref,
        out_ref,
        cnt_vmem,
        cnt_smem,
        cnt_send_sems,
        cnt_recv_sems,
        d_send_sems,
        d_recv_sems,
        copy_sem,
    ):
        me = lax.axis_index("i")

        barrier = pltpu.get_barrier_semaphore()
        for o in range(1, N_DEV):
            pl.semaphore_signal(
                barrier,
                inc=1,
                device_id=((me + o) % N_DEV,),
                device_id_type=pl.DeviceIdType.MESH,
            )
        pl.semaphore_wait(barrier, N_DEV - 1)

        own = pltpu.make_async_copy(
            counts_ref, cnt_vmem.at[pl.ds(me, 1)], copy_sem
        )
        own.start()
        own.wait()
        for o in range(1, N_DEV):
            rdma = pltpu.make_async_remote_copy(
                src_ref=counts_ref,
                dst_ref=cnt_vmem.at[pl.ds(me, 1)],
                send_sem=cnt_send_sems.at[o],
                recv_sem=cnt_recv_sems.at[o],
                device_id=((me + o) % N_DEV,),
                device_id_type=pl.DeviceIdType.MESH,
            )
            rdma.start()
        for o in range(1, N_DEV):
            pltpu.make_async_remote_copy(
                src_ref=counts_ref,
                dst_ref=cnt_vmem.at[pl.ds(0, 1)],
                send_sem=cnt_send_sems.at[o],
                recv_sem=cnt_recv_sems.at[o],
                device_id=(0,),
                device_id_type=pl.DeviceIdType.MESH,
            ).wait_recv()
        for o in range(1, N_DEV):
            pltpu.make_async_remote_copy(
                src_ref=counts_ref,
                dst_ref=cnt_vmem.at[pl.ds(0, 1)],
                send_sem=cnt_send_sems.at[o],
                recv_sem=cnt_recv_sems.at[o],
                device_id=(0,),
                device_id_type=pl.DeviceIdType.MESH,
            ).wait_send()

        cp = pltpu.make_async_copy(cnt_vmem, cnt_smem, copy_sem)
        cp.start()
        cp.wait()

        def cnt(s, d):
            return cnt_smem[s, d]

        def col_off_before(src, dst):
            acc = jnp.int32(0)
            for s in range(N_DEV):
                acc = acc + jnp.where(s < src, cnt(s, dst), 0)
            return acc

        def row_off_before(src, dst):
            acc = jnp.int32(0)
            for d in range(N_DEV):
                acc = acc + jnp.where(d < dst, cnt(src, d), 0)
            return acc

        for o in range(1, N_DEV):
            dst_rank = (me + o) % N_DEV
            c = cnt(me, dst_rank)
            src0 = row_off_before(me, dst_rank)
            dst0 = col_off_before(me, dst_rank)
            done = jnp.int32(0)
            for k in range(NBITS - 1, -1, -1):
                size = 1 << k
                bit = (c // size) % 2

                @pl.when(bit == 1)
                def _(o=o, k=k, size=size, src0=src0, dst0=dst0, done=done,
                      dst_rank=dst_rank):
                    pltpu.make_async_remote_copy(
                        src_ref=rows(x_ref, src0 + done, size),
                        dst_ref=rows(out_ref, dst0 + done, size),
                        send_sem=d_send_sems.at[o, k],
                        recv_sem=d_recv_sems.at[o, k],
                        device_id=(dst_rank,),
                        device_id_type=pl.DeviceIdType.MESH,
                    ).start()

                done = done + bit * size

        c = cnt(me, me)
        src0 = row_off_before(me, me)
        dst0 = col_off_before(me, me)
        done = jnp.int32(0)
        for k in range(NBITS - 1, -1, -1):
            size = 1 << k
            bit = (c // size) % 2

            @pl.when(bit == 1)
            def _(size=size, src0=src0, dst0=dst0, done=done):
                local = pltpu.make_async_copy(
                    rows(x_ref, src0 + done, size),
                    rows(out_ref, dst0 + done, size),
                    copy_sem,
                )
                local.start()
                local.wait()

            done = done + bit * size

        for o in range(1, N_DEV):
            src_rank = (me - o) % N_DEV
            c_in = cnt(src_rank, me)
            for k in range(NBITS - 1, -1, -1):
                size = 1 << k
                bit = (c_in // size) % 2

                @pl.when(bit == 1)
                def _(o=o, k=k, size=size):
                    pltpu.make_async_remote_copy(
                        src_ref=rows(x_ref, 0, size),
                        dst_ref=rows(out_ref, 0, size),
                        send_sem=d_send_sems.at[o, k],
                        recv_sem=d_recv_sems.at[o, k],
                        device_id=(0,),
                        device_id_type=pl.DeviceIdType.MESH,
                    ).wait_recv()

        for o in range(1, N_DEV):
            dst_rank = (me + o) % N_DEV
            c_out = cnt(me, dst_rank)
            for k in range(NBITS - 1, -1, -1):
                size = 1 << k
                bit = (c_out // size) % 2

                @pl.when(bit == 1)
                def _(o=o, k=k, size=size):
                    pltpu.make_async_remote_copy(
                        src_ref=rows(x_ref, 0, size),
                        dst_ref=rows(out_ref, 0, size),
                        send_sem=d_send_sems.at[o, k],
                        recv_sem=d_recv_sems.at[o, k],
                        device_id=(0,),
                        device_id_type=pl.DeviceIdType.MESH,
                    ).wait_send()

    out = pl.pallas_call(
        body,
        out_shape=jax.ShapeDtypeStruct((t * d_model,), jnp.float32),
        in_specs=[
            pl.BlockSpec(memory_space=pltpu.VMEM),
            pl.BlockSpec(memory_space=pltpu.VMEM),
        ],
        out_specs=pl.BlockSpec(memory_space=pltpu.VMEM),
        scratch_shapes=[
            pltpu.VMEM((N_DEV, N_DEV), jnp.int32),
            pltpu.SMEM((N_DEV, N_DEV), jnp.int32),
            pltpu.SemaphoreType.DMA((N_DEV,)),
            pltpu.SemaphoreType.DMA((N_DEV,)),
            pltpu.SemaphoreType.DMA((N_DEV, NBITS)),
            pltpu.SemaphoreType.DMA((N_DEV, NBITS)),
            pltpu.SemaphoreType.DMA,
        ],
        compiler_params=pltpu.CompilerParams(collective_id=0),
    )(x_sorted, counts)
    return out.reshape(t, d_model)


# device time: 22408 ns/iter; 1.8558x vs baseline; 1.8558x over previous
import jax
import jax.numpy as jnp
from jax import lax
from jax.experimental import pallas as pl
from jax.experimental.pallas import tpu as pltpu

N_DEV = 8
NBITS = 10


def kernel(x, dest):
    t, d_model = x.shape

    order = jnp.argsort(dest, stable=True)
    x_sorted = jnp.take(x, order, axis=0).reshape(-1)
    counts = (
        (dest[:, None] == jnp.arange(N_DEV, dtype=dest.dtype)[None, :])
        .astype(jnp.int32)
        .sum(axis=0)
        .reshape(1, N_DEV)
    )

    def rows(ref, row0, nrows):
        if isinstance(row0, int):
            return ref.at[pl.ds(row0 * d_model, nrows * d_model)]
        return ref.at[pl.ds(pl.multiple_of(row0 * d_model, d_model),
                            nrows * d_model)]

    def body(
        x_ref,
        counts_ref,
        out_ref,
        cnt_vmem,
        cnt_smem,
        cnt_send_sems,
        cnt_recv_sems,
        d_send_sems,
        d_recv_sems,
        copy_sem,
    ):
        me = lax.axis_index("i")

        barrier = pltpu.get_barrier_semaphore()
        for o in range(1, N_DEV):
            pl.semaphore_signal(
                barrier,
                inc=1,
                device_id=((me + o) % N_DEV,),
                device_id_type=pl.DeviceIdType.MESH,
            )
        pl.semaphore_wait(barrier, N_DEV - 1)

        own = pltpu.make_async_copy(
            counts_ref, cnt_vmem.at[pl.ds(me, 1)], copy_sem
        )
        own.start()
        own.wait()
        for o in range(1, N_DEV):
            rdma = pltpu.make_async_remote_copy(
                src_ref=counts_ref,
                dst_ref=cnt_vmem.at[pl.ds(me, 1)],
                send_sem=cnt_send_sems.at[o],
                recv_sem=cnt_recv_sems.at[o],
                device_id=((me + o) % N_DEV,),
                device_id_type=pl.DeviceIdType.MESH,
            )
            rdma.start()
        for o in range(1, N_DEV):
            pltpu.make_async_remote_copy(
                src_ref=counts_ref,
                dst_ref=cnt_vmem.at[pl.ds(0, 1)],
                send_sem=cnt_send_sems.at[o],
                recv_sem=cnt_recv_sems.at[o],
                device_id=(0,),
                device_id_type=pl.DeviceIdType.MESH,
            ).wait_recv()
        for o in range(1, N_DEV):
            pltpu.make_async_remote_copy(
                src_ref=counts_ref,
                dst_ref=cnt_vmem.at[pl.ds(0, 1)],
                send_sem=cnt_send_sems.at[o],
                recv_sem=cnt_recv_sems.at[o],
                device_id=(0,),
                device_id_type=pl.DeviceIdType.MESH,
            ).wait_send()

        cp = pltpu.make_async_copy(cnt_vmem, cnt_smem, copy_sem)
        cp.start()
        cp.wait()

        def cnt(s, d):
            return cnt_smem[s, d]

        def col_off_before(src, dst):
            acc = jnp.int32(0)
            for s in range(N_DEV):
                acc = acc + jnp.where(s < src, cnt(s, dst), 0)
            return acc

        def row_off_before(src, dst):
            acc = jnp.int32(0)
            for d in range(N_DEV):
                acc = acc + jnp.where(d < dst, cnt(src, d), 0)
            return acc

        for o in range(1, N_DEV):
            dst_rank = (me + o) % N_DEV
            c = cnt(me, dst_rank)
            src0 = row_off_before(me, dst_rank)
            dst0 = col_off_before(me, dst_rank)
            done = jnp.int32(0)
            for k in range(NBITS - 1, -1, -1):
                size = 1 << k
                bit = (c // size) % 2

                @pl.when(bit == 1)
                def _(o=o, k=k, size=size, src0=src0, dst0=dst0, done=done,
                      dst_rank=dst_rank):
                    pltpu.make_async_remote_copy(
                        src_ref=rows(x_ref, src0 + done, size),
                        dst_ref=rows(out_ref, dst0 + done, size),
                        send_sem=d_send_sems.at[o, k],
                        recv_sem=d_recv_sems.at[o, k],
                        device_id=(dst_rank,),
                        device_id_type=pl.DeviceIdType.MESH,
                    ).start()

                done = done + bit * size

        c = cnt(me, me)
        src0 = row_off_before(me, me)
        dst0 = col_off_before(me, me)
        done = jnp.int32(0)
        for k in range(NBITS - 1, -1, -1):
            size = 1 << k
            bit = (c // size) % 2

            @pl.when(bit == 1)
            def _(size=size, src0=src0, dst0=dst0, done=done):
                local = pltpu.make_async_copy(
                    rows(x_ref, src0 + done, size),
                    rows(out_ref, dst0 + done, size),
                    copy_sem,
                )
                local.start()
                local.wait()

            done = done + bit * size

        for o in range(1, N_DEV):
            src_rank = (me - o) % N_DEV
            c_in = cnt(src_rank, me)
            for k in range(NBITS - 1, -1, -1):
                size = 1 << k
                bit = (c_in // size) % 2

                @pl.when(bit == 1)
                def _(o=o, k=k, size=size):
                    pltpu.make_async_remote_copy(
                        src_ref=rows(x_ref, 0, size),
                        dst_ref=rows(out_ref, 0, size),
                        send_sem=d_send_sems.at[o, k],
                        recv_sem=d_recv_sems.at[o, k],
                        device_id=(0,),
                        device_id_type=pl.DeviceIdType.MESH,
                    ).wait_recv()

        for o in range(1, N_DEV):
            dst_rank = (me + o) % N_DEV
            c_out = cnt(me, dst_rank)
            for k in range(NBITS - 1, -1, -1):
                size = 1 << k
                bit = (c_out // size) % 2

                @pl.when(bit == 1)
                def _(o=o, k=k, size=size):
                    pltpu.make_async_remote_copy(
                        src_ref=rows(x_ref, 0, size),
                        dst_ref=rows(out_ref, 0, size),
                        send_sem=d_send_sems.at[o, k],
                        recv_sem=d_recv_sems.at[o, k],
                        device_id=(0,),
                        device_id_type=pl.DeviceIdType.MESH,
                    ).wait_send()

    out = pl.pallas_call(
        body,
        out_shape=jax.ShapeDtypeStruct((t * d_model,), jnp.float32),
        in_specs=[
            pl.BlockSpec(memory_space=pltpu.VMEM),
            pl.BlockSpec(memory_space=pltpu.VMEM),
        ],
        out_specs=pl.BlockSpec(memory_space=pltpu.VMEM),
        scratch_shapes=[
            pltpu.VMEM((N_DEV, N_DEV), jnp.int32),
            pltpu.SMEM((N_DEV, N_DEV), jnp.int32),
            pltpu.SemaphoreType.DMA((N_DEV,)),
            pltpu.SemaphoreType.DMA((N_DEV,)),
            pltpu.SemaphoreType.DMA((N_DEV, NBITS)),
            pltpu.SemaphoreType.DMA((N_DEV, NBITS)),
            pltpu.SemaphoreType.DMA,
        ],
        compiler_params=pltpu.CompilerParams(collective_id=0),
    )(x_sorted, counts)
    return out.reshape(t, d_model)


# device time: 19983 ns/iter; 2.0810x vs baseline; 1.1214x over previous
import jax
import jax.numpy as jnp
from jax import lax
from jax.experimental import pallas as pl
from jax.experimental.pallas import tpu as pltpu

N_DEV = 8
NBITS = 10


def kernel(x, dest):
    t, d_model = x.shape

    oh = (dest[:, None] == jnp.arange(N_DEV, dtype=dest.dtype)[None, :]).astype(
        jnp.int32
    )
    counts_1d = oh.sum(axis=0)
    counts = counts_1d.reshape(1, N_DEV).astype(jnp.int32)
    lstart = jnp.concatenate(
        [jnp.zeros((1,), jnp.int32), jnp.cumsum(counts_1d)[:-1]]
    )
    pos = ((jnp.cumsum(oh, axis=0) - oh) * oh).sum(axis=1)
    target = pos + (lstart[None, :] * oh).sum(axis=1)
    perm = (
        target[None, :] == jnp.arange(t, dtype=jnp.int32)[:, None]
    ).astype(x.dtype)
    x_sorted = jax.lax.dot(
        perm, x, precision=jax.lax.Precision.HIGHEST
    ).reshape(-1)

    def rows(ref, row0, nrows):
        if isinstance(row0, int):
            return ref.at[pl.ds(row0 * d_model, nrows * d_model)]
        return ref.at[pl.ds(pl.multiple_of(row0 * d_model, d_model),
                            nrows * d_model)]

    def body(
        x_ref,
        counts_ref,
        out_ref,
        cnt_vmem,
        cnt_smem,
        cnt_send_sems,
        cnt_recv_sems,
        d_send_sems,
        d_recv_sems,
        copy_sem,
    ):
        me = lax.axis_index("i")

        barrier = pltpu.get_barrier_semaphore()
        for o in range(1, N_DEV):
            pl.semaphore_signal(
                barrier,
                inc=1,
                device_id=((me + o) % N_DEV,),
                device_id_type=pl.DeviceIdType.MESH,
            )
        pl.semaphore_wait(barrier, N_DEV - 1)

        own = pltpu.make_async_copy(
            counts_ref, cnt_vmem.at[pl.ds(me, 1)], copy_sem
        )
        own.start()
        own.wait()
        for o in range(1, N_DEV):
            rdma = pltpu.make_async_remote_copy(
                src_ref=counts_ref,
                dst_ref=cnt_vmem.at[pl.ds(me, 1)],
                send_sem=cnt_send_sems.at[o],
                recv_sem=cnt_recv_sems.at[o],
                device_id=((me + o) % N_DEV,),
                device_id_type=pl.DeviceIdType.MESH,
            )
            rdma.start()
        for o in range(1, N_DEV):
            pltpu.make_async_remote_copy(
                src_ref=counts_ref,
                dst_ref=cnt_vmem.at[pl.ds(0, 1)],
                send_sem=cnt_send_sems.at[o],
                recv_sem=cnt_recv_sems.at[o],
                device_id=(0,),
                device_id_type=pl.DeviceIdType.MESH,
            ).wait_recv()
        for o in range(1, N_DEV):
            pltpu.make_async_remote_copy(
                src_ref=counts_ref,
                dst_ref=cnt_vmem.at[pl.ds(0, 1)],
                send_sem=cnt_send_sems.at[o],
                recv_sem=cnt_recv_sems.at[o],
                device_id=(0,),
                device_id_type=pl.DeviceIdType.MESH,
            ).wait_send()

        cp = pltpu.make_async_copy(cnt_vmem, cnt_smem, copy_sem)
        cp.start()
        cp.wait()

        def cnt(s, d):
            return cnt_smem[s, d]

        def col_off_before(src, dst):
            acc = jnp.int32(0)
            for s in range(N_DEV):
                acc = acc + jnp.where(s < src, cnt(s, dst), 0)
            return acc

        def row_off_before(src, dst):
            acc = jnp.int32(0)
            for d in range(N_DEV):
                acc = acc + jnp.where(d < dst, cnt(src, d), 0)
            return acc

        for o in range(1, N_DEV):
            dst_rank = (me + o) % N_DEV
            c = cnt(me, dst_rank)
            src0 = row_off_before(me, dst_rank)
            dst0 = col_off_before(me, dst_rank)
            done = jnp.int32(0)
            for k in range(NBITS - 1, -1, -1):
                size = 1 << k
                bit = (c // size) % 2

                @pl.when(bit == 1)
                def _(o=o, k=k, size=size, src0=src0, dst0=dst0, done=done,
                      dst_rank=dst_rank):
                    pltpu.make_async_remote_copy(
                        src_ref=rows(x_ref, src0 + done, size),
                        dst_ref=rows(out_ref, dst0 + done, size),
                        send_sem=d_send_sems.at[o, k],
                        recv_sem=d_recv_sems.at[o, k],
                        device_id=(dst_rank,),
                        device_id_type=pl.DeviceIdType.MESH,
                    ).start()

                done = done + bit * size

        c = cnt(me, me)
        src0 = row_off_before(me, me)
        dst0 = col_off_before(me, me)
        done = jnp.int32(0)
        for k in range(NBITS - 1, -1, -1):
            size = 1 << k
            bit = (c // size) % 2

            @pl.when(bit == 1)
            def _(size=size, src0=src0, dst0=dst0, done=done):
                local = pltpu.make_async_copy(
                    rows(x_ref, src0 + done, size),
                    rows(out_ref, dst0 + done, size),
                    copy_sem,
                )
                local.start()
                local.wait()

            done = done + bit * size

        for o in range(1, N_DEV):
            src_rank = (me - o) % N_DEV
            c_in = cnt(src_rank, me)
            for k in range(NBITS - 1, -1, -1):
                size = 1 << k
                bit = (c_in // size) % 2

                @pl.when(bit == 1)
                def _(o=o, k=k, size=size):
                    pltpu.make_async_remote_copy(
                        src_ref=rows(x_ref, 0, size),
                        dst_ref=rows(out_ref, 0, size),
                        send_sem=d_send_sems.at[o, k],
                        recv_sem=d_recv_sems.at[o, k],
                        device_id=(0,),
                        device_id_type=pl.DeviceIdType.MESH,
                    ).wait_recv()

        for o in range(1, N_DEV):
            dst_rank = (me + o) % N_DEV
            c_out = cnt(me, dst_rank)
            for k in range(NBITS - 1, -1, -1):
                size = 1 << k
                bit = (c_out // size) % 2

                @pl.when(bit == 1)
                def _(o=o, k=k, size=size):
                    pltpu.make_async_remote_copy(
                        src_ref=rows(x_ref, 0, size),
                        dst_ref=rows(out_ref, 0, size),
                        send_sem=d_send_sems.at[o, k],
                        recv_sem=d_recv_sems.at[o, k],
                        device_id=(0,),
                        device_id_type=pl.DeviceIdType.MESH,
                    ).wait_send()

    out = pl.pallas_call(
        body,
        out_shape=jax.ShapeDtypeStruct((t * d_model,), jnp.float32),
        in_specs=[
            pl.BlockSpec(memory_space=pltpu.VMEM),
            pl.BlockSpec(memory_space=pltpu.VMEM),
        ],
        out_specs=pl.BlockSpec(memory_space=pltpu.VMEM),
        scratch_shapes=[
            pltpu.VMEM((N_DEV, N_DEV), jnp.int32),
            pltpu.SMEM((N_DEV, N_DEV), jnp.int32),
            pltpu.SemaphoreType.DMA((N_DEV,)),
            pltpu.SemaphoreType.DMA((N_DEV,)),
            pltpu.SemaphoreType.DMA((N_DEV, NBITS)),
            pltpu.SemaphoreType.DMA((N_DEV, NBITS)),
            pltpu.SemaphoreType.DMA,
        ],
        compiler_params=pltpu.CompilerParams(collective_id=0),
    )(x_sorted, counts)
    return out.reshape(t, d_model)
